# baseline (speedup 1.0000x reference)
import functools
import math

import jax
import jax.numpy as jnp
import numpy as np
from jax import lax
from jax.experimental import pallas as pl
from jax.experimental.pallas import tpu as pltpu
from jax.experimental.pallas import tpu_sc as plsc

B = 16
CIN = 768
COUT = 64
K = 8192
H = 16
W = 16
T = 256
SY = 8
SX = 32
NTOK = B * T
CT = CIN
KC = 1024
KT = 1024
MASK_PROB = 0.015

SC_NC = 2
SC_NS = 16
SC_NW = SC_NC * SC_NS
TOK_PER_W = NTOK // SC_NW


def _pos_tokens() -> np.ndarray:
    mask = np.ones((1, H, W), np.float64)
    y_embed = np.cumsum(mask, axis=1)
    x_embed = np.cumsum(mask, axis=2)
    eps = 1e-6
    y_embed = y_embed / (y_embed[:, -1:, :] + eps) * 2 * math.pi
    x_embed = x_embed / (x_embed[:, :, -1:] + eps) * 2 * math.pi
    d = COUT // 2
    dim_t = np.arange(d, dtype=np.float64)
    dim_t = 10000.0 ** (2 * np.floor(dim_t / 2) / d)
    px = x_embed[:, :, :, None] / dim_t
    py = y_embed[:, :, :, None] / dim_t
    px = np.stack((np.sin(px[:, :, :, 0::2]), np.cos(px[:, :, :, 1::2])),
                  axis=4).reshape(1, H, W, -1)
    py = np.stack((np.sin(py[:, :, :, 0::2]), np.cos(py[:, :, :, 1::2])),
                  axis=4).reshape(1, H, W, -1)
    pos = np.concatenate((py, px), axis=3)
    return pos.reshape(T, COUT).astype(np.float32)


_POS = _pos_tokens()
np.random.seed(0)
_TMP = int(np.random.randint(H * W))


def _codebook_xform_body(cb_ref, wp_ref, bp_ref, t_ref, cbe_ref):
    cb = cb_ref[...]
    t = (jnp.dot(cb, wp_ref[...],
                 preferred_element_type=jnp.float32) + bp_ref[0])
    t_ref[...] = jnp.concatenate(
        [t, jnp.zeros((KT, COUT), jnp.float32)], axis=-1)
    cn = jnp.sum(cb * cb, axis=-1, keepdims=True)
    cbe_ref[...] = jnp.concatenate([-2.0 * cb, cn], axis=-1)


def _codebook_xform(codebook, Wp, bp):
    return pl.pallas_call(
        _codebook_xform_body,
        grid=(K // KT,),
        in_specs=[
            pl.BlockSpec((KT, COUT), lambda k: (k, 0)),
            pl.BlockSpec((COUT, COUT), lambda k: (0, 0)),
            pl.BlockSpec((1, COUT), lambda k: (0, 0)),
        ],
        out_specs=[
            pl.BlockSpec((KT, 2 * COUT), lambda k: (k, 0)),
            pl.BlockSpec((KT, COUT + 1), lambda k: (k, 0)),
        ],
        out_shape=[
            jax.ShapeDtypeStruct((K, 2 * COUT), jnp.float32),
            jax.ShapeDtypeStruct((K, COUT + 1), jnp.float32),
        ],
    )(codebook, Wp, bp.reshape(1, COUT))


def _encode_vq_body(img_ref, w1_ref, b1_ref, g1_ref, be1_ref, cbe_ref,
                    h_ref, idx_ref):
    x = img_ref[0]
    m1 = jnp.maximum(x, pltpu.roll(x, 96, axis=2))
    m2 = jnp.maximum(m1, pltpu.roll(m1, 127, axis=2))
    li = lax.broadcasted_iota(jnp.int32, (128, 32), 0)
    ji = lax.broadcasted_iota(jnp.int32, (128, 32), 1)
    sel = (li == 2 * ji + 32 * (ji // 16)).astype(jnp.float32)
    p = lax.dot_general(m2, sel, (((2,), (0,)), ((), ())),
                        preferred_element_type=jnp.float32)
    part = lax.dot_general(p, w1_ref[...], (((0,), (0,)), ((), ())),
                           preferred_element_type=jnp.float32)

    h1 = part + b1_ref[0]
    m = jnp.mean(h1, axis=-1, keepdims=True)
    v = jnp.mean((h1 - m) ** 2, axis=-1, keepdims=True)
    h1 = (h1 - m) * lax.rsqrt(v + 1e-5) * g1_ref[0] + be1_ref[0]
    h1 = jnp.maximum(h1, 0.0)
    h_ref[0] = h1
    hp = jnp.concatenate(
        [h1.reshape(T, COUT), jnp.ones((T, 1), jnp.float32)],
        axis=-1)
    li = lax.broadcasted_iota(jnp.int32, (T, KC), 1)

    def vq_step(kc, carry):
        bv, bi = carry
        off = pl.multiple_of(kc * KC, KC)
        cb = cbe_ref[pl.ds(off, KC), :]
        s = lax.dot_general(hp, cb, (((1,), (1,)), ((), ())),
                            preferred_element_type=jnp.float32)
        mv = jnp.min(s, axis=-1, keepdims=True)
        ai = jnp.min(jnp.where(s == mv, li, KC), axis=-1,
                     keepdims=True)
        gi = ai + kc * KC
        upd = mv < bv
        return jnp.where(upd, mv, bv), jnp.where(upd, gi, bi)

    bv0 = jnp.full((T, 1), jnp.inf, jnp.float32)
    bi0 = jnp.zeros((T, 1), jnp.int32)
    _, bi = lax.fori_loop(0, K // KC, vq_step, (bv0, bi0))
    idx_ref[0] = bi


def _encode_vq(img4, W1, b1, g1, be1, cbext):
    nb = img4.shape[0]
    return pl.pallas_call(
        _encode_vq_body,
        grid=(nb,),
        in_specs=[
            pl.BlockSpec((1, CT, SY, 128), lambda b: (b, 0, 0, 0)),
            pl.BlockSpec((CT, COUT), lambda b: (0, 0)),
            pl.BlockSpec((1, COUT), lambda b: (0, 0)),
            pl.BlockSpec((1, COUT), lambda b: (0, 0)),
            pl.BlockSpec((1, COUT), lambda b: (0, 0)),
            pl.BlockSpec((K, COUT + 1), lambda b: (0, 0)),
        ],
        out_specs=[
            pl.BlockSpec((1, SY, SX, COUT), lambda b: (b, 0, 0, 0)),
            pl.BlockSpec((1, T, 1), lambda b: (b, 0, 0)),
        ],
        out_shape=[
            jax.ShapeDtypeStruct((nb, SY, SX, COUT), jnp.float32),
            jax.ShapeDtypeStruct((nb, T, 1), jnp.int32),
        ],
        compiler_params=pltpu.CompilerParams(
            fuse_transposed_lhs_in_matmul=True),
    )(img4, W1, b1.reshape(1, COUT), g1.reshape(1, COUT),
      be1.reshape(1, COUT), cbext)


def _gather_rows(table, idx):
    mesh = plsc.VectorSubcoreMesh(core_axis_name="c", subcore_axis_name="s")
    n = idx.shape[0]
    tok_per_w = n // SC_NW

    @functools.partial(
        pl.kernel, mesh=mesh,
        out_type=jax.ShapeDtypeStruct((n, 2 * COUT), jnp.float32),
        scratch_types=[
            pltpu.VMEM((tok_per_w,), jnp.int32),
            pltpu.VMEM((tok_per_w, 2 * COUT), jnp.float32),
            pltpu.SemaphoreType.DMA,
        ],
    )
    def k(table_hbm, idx_hbm, out_hbm, idx_v, rows_v, sem):
        wid = lax.axis_index("s") * SC_NC + lax.axis_index("c")
        base = wid * tok_per_w
        pltpu.sync_copy(idx_hbm.at[pl.ds(base, tok_per_w)], idx_v)
        pltpu.async_copy(table_hbm.at[idx_v], rows_v, sem).wait()
        pltpu.sync_copy(rows_v, out_hbm.at[pl.ds(base, tok_per_w)])

    return k(table, idx)


def _finish_body(q_ref, h_ref, idx_ref, bern_ref, pos_ref, me_ref,
                 g2_ref, be2_ref, xqo_ref, lab_ref):
    idxv = idx_ref[0]
    bern = bern_ref[0]
    ti = lax.broadcasted_iota(jnp.int32, (T, 1), 0)
    tl = jnp.sum(jnp.where(ti == _TMP, idxv, 0), axis=0, keepdims=True)
    msk = (idxv == tl) & (bern > 0.5)
    mf = msk.astype(jnp.float32)
    emb = q_ref[0][:, :COUT] + h_ref[0]
    emb = emb * (1.0 - mf) + me_ref[0] * mf
    emb = emb + pos_ref[...]
    m = jnp.mean(emb, axis=-1, keepdims=True)
    v = jnp.mean((emb - m) ** 2, axis=-1, keepdims=True)
    xqo_ref[0] = (emb - m) * lax.rsqrt(v + 1e-5) * g2_ref[0] + be2_ref[0]
    lab_ref[0] = jnp.where(msk, idxv, -100)


def _finish(q3, h3, idx3, bern3, mask_emb, g2, be2):
    nb = q3.shape[0]
    return pl.pallas_call(
        _finish_body,
        grid=(nb,),
        in_specs=[
            pl.BlockSpec((1, T, 2 * COUT), lambda b: (b, 0, 0)),
            pl.BlockSpec((1, T, COUT), lambda b: (b, 0, 0)),
            pl.BlockSpec((1, T, 1), lambda b: (b, 0, 0)),
            pl.BlockSpec((1, T, 1), lambda b: (b, 0, 0)),
            pl.BlockSpec((T, COUT), lambda b: (0, 0)),
            pl.BlockSpec((1, COUT), lambda b: (0, 0)),
            pl.BlockSpec((1, COUT), lambda b: (0, 0)),
            pl.BlockSpec((1, COUT), lambda b: (0, 0)),
        ],
        out_specs=[
            pl.BlockSpec((1, T, COUT), lambda b: (b, 0, 0)),
            pl.BlockSpec((1, T, 1), lambda b: (b, 0, 0)),
        ],
        out_shape=[
            jax.ShapeDtypeStruct((nb, T, COUT), jnp.float32),
            jax.ShapeDtypeStruct((nb, T, 1), jnp.int32),
        ],
    )(q3, h3, idx3, bern3, jnp.asarray(_POS), mask_emb,
      g2.reshape(1, COUT), be2.reshape(1, COUT))


def kernel(img, W1, b1, g1, be1, codebook, Wp, bp, mask_emb, g2, be2):
    img4 = img.reshape(B, CIN, SY, 128)

    tbl, cbext = _codebook_xform(codebook, Wp, bp)

    bern = jax.random.bernoulli(jax.random.key(42), MASK_PROB,
                                (B, 1, 1)).astype(jnp.float32)
    bern3 = jnp.broadcast_to(bern, (B, T, 1))

    hb = B // 2
    xqos, labs = [], []
    for i in range(2):
        sl = slice(i * hb, (i + 1) * hb)
        h4, idx3d = _encode_vq(img4[sl], W1, b1, g1, be1, cbext)
        q = _gather_rows(tbl, idx3d.reshape(hb * T))
        xqo_h, lab_h = _finish(q.reshape(hb, T, 2 * COUT),
                               h4.reshape(hb, T, COUT), idx3d,
                               bern3[sl], mask_emb, g2, be2)
        xqos.append(xqo_h)
        labs.append(lab_h)

    xqo = jnp.concatenate(xqos, axis=0)
    lab = jnp.concatenate(labs, axis=0)
    vm = jnp.ones((B, T), jnp.int32)
    return (xqo, vm, lab.reshape(B, T))


# speedup vs baseline: 1.1733x; 1.1733x over previous
import functools
import math

import jax
import jax.numpy as jnp
import numpy as np
from jax import lax
from jax.experimental import pallas as pl
from jax.experimental.pallas import tpu as pltpu
from jax.experimental.pallas import tpu_sc as plsc

B = 16
CIN = 768
COUT = 64
K = 8192
H = 16
W = 16
T = 256
SY = 8
SX = 32
NTOK = B * T
CT = CIN
KC = 1024
KT = 1024
MASK_PROB = 0.015

SC_NC = 2
SC_NS = 16
SC_NW = SC_NC * SC_NS
TOK_PER_W = NTOK // SC_NW


def _pos_tokens() -> np.ndarray:
    mask = np.ones((1, H, W), np.float64)
    y_embed = np.cumsum(mask, axis=1)
    x_embed = np.cumsum(mask, axis=2)
    eps = 1e-6
    y_embed = y_embed / (y_embed[:, -1:, :] + eps) * 2 * math.pi
    x_embed = x_embed / (x_embed[:, :, -1:] + eps) * 2 * math.pi
    d = COUT // 2
    dim_t = np.arange(d, dtype=np.float64)
    dim_t = 10000.0 ** (2 * np.floor(dim_t / 2) / d)
    px = x_embed[:, :, :, None] / dim_t
    py = y_embed[:, :, :, None] / dim_t
    px = np.stack((np.sin(px[:, :, :, 0::2]), np.cos(px[:, :, :, 1::2])),
                  axis=4).reshape(1, H, W, -1)
    py = np.stack((np.sin(py[:, :, :, 0::2]), np.cos(py[:, :, :, 1::2])),
                  axis=4).reshape(1, H, W, -1)
    pos = np.concatenate((py, px), axis=3)
    return pos.reshape(T, COUT).astype(np.float32)


_POS = _pos_tokens()
np.random.seed(0)
_TMP = int(np.random.randint(H * W))


def _codebook_xform_body(cb_ref, wp_ref, bp_ref, t_ref, cbe_ref):
    cb = cb_ref[...]
    t = (jnp.dot(cb, wp_ref[...],
                 preferred_element_type=jnp.float32) + bp_ref[0])
    t_ref[...] = jnp.concatenate(
        [t, jnp.zeros((KT, COUT), jnp.float32)], axis=-1)
    cn = jnp.sum(cb * cb, axis=-1, keepdims=True)
    cbe_ref[...] = jnp.concatenate([-2.0 * cb, cn], axis=-1)


def _codebook_xform(codebook, Wp, bp):
    return pl.pallas_call(
        _codebook_xform_body,
        grid=(K // KT,),
        in_specs=[
            pl.BlockSpec((KT, COUT), lambda k: (k, 0)),
            pl.BlockSpec((COUT, COUT), lambda k: (0, 0)),
            pl.BlockSpec((1, COUT), lambda k: (0, 0)),
        ],
        out_specs=[
            pl.BlockSpec((KT, 2 * COUT), lambda k: (k, 0)),
            pl.BlockSpec((KT, COUT + 1), lambda k: (k, 0)),
        ],
        out_shape=[
            jax.ShapeDtypeStruct((K, 2 * COUT), jnp.float32),
            jax.ShapeDtypeStruct((K, COUT + 1), jnp.float32),
        ],
    )(codebook, Wp, bp.reshape(1, COUT))


def _encode_vq_body(img_ref, w1_ref, b1_ref, g1_ref, be1_ref, cbe_ref,
                    h_ref, idx_ref):
    x = img_ref[0]
    m1 = jnp.maximum(x, pltpu.roll(x, 96, axis=2))
    m2 = jnp.maximum(m1, pltpu.roll(m1, 127, axis=2))
    li = lax.broadcasted_iota(jnp.int32, (128, 32), 0)
    ji = lax.broadcasted_iota(jnp.int32, (128, 32), 1)
    sel = (li == 2 * ji + 32 * (ji // 16)).astype(jnp.float32)
    p = lax.dot_general(m2, sel, (((2,), (0,)), ((), ())),
                        preferred_element_type=jnp.float32)
    part = lax.dot_general(p, w1_ref[...], (((0,), (0,)), ((), ())),
                           preferred_element_type=jnp.float32)

    h1 = part + b1_ref[0]
    m = jnp.mean(h1, axis=-1, keepdims=True)
    v = jnp.mean((h1 - m) ** 2, axis=-1, keepdims=True)
    h1 = (h1 - m) * lax.rsqrt(v + 1e-5) * g1_ref[0] + be1_ref[0]
    h1 = jnp.maximum(h1, 0.0)
    h_ref[0] = h1
    hp = jnp.concatenate(
        [h1.reshape(T, COUT), jnp.ones((T, 1), jnp.float32)],
        axis=-1)
    li = lax.broadcasted_iota(jnp.int32, (T, KC), 1)

    def vq_step(kc, carry):
        bv, bi = carry
        off = pl.multiple_of(kc * KC, KC)
        cb = cbe_ref[pl.ds(off, KC), :]
        s = lax.dot_general(hp, cb, (((1,), (1,)), ((), ())),
                            preferred_element_type=jnp.float32)
        mv = jnp.min(s, axis=-1, keepdims=True)
        ai = jnp.min(jnp.where(s == mv, li, KC), axis=-1,
                     keepdims=True)
        gi = ai + kc * KC
        upd = mv < bv
        return jnp.where(upd, mv, bv), jnp.where(upd, gi, bi)

    bv0 = jnp.full((T, 1), jnp.inf, jnp.float32)
    bi0 = jnp.zeros((T, 1), jnp.int32)
    _, bi = lax.fori_loop(0, K // KC, vq_step, (bv0, bi0))
    idx_ref[0] = bi


def _encode_vq(img4, W1, b1, g1, be1, cbext):
    nb = img4.shape[0]
    return pl.pallas_call(
        _encode_vq_body,
        grid=(nb,),
        in_specs=[
            pl.BlockSpec((1, CT, SY, 128), lambda b: (b, 0, 0, 0)),
            pl.BlockSpec((CT, COUT), lambda b: (0, 0)),
            pl.BlockSpec((1, COUT), lambda b: (0, 0)),
            pl.BlockSpec((1, COUT), lambda b: (0, 0)),
            pl.BlockSpec((1, COUT), lambda b: (0, 0)),
            pl.BlockSpec((K, COUT + 1), lambda b: (0, 0)),
        ],
        out_specs=[
            pl.BlockSpec((1, SY, SX, COUT), lambda b: (b, 0, 0, 0)),
            pl.BlockSpec((1, T, 1), lambda b: (b, 0, 0)),
        ],
        out_shape=[
            jax.ShapeDtypeStruct((nb, SY, SX, COUT), jnp.float32),
            jax.ShapeDtypeStruct((nb, T, 1), jnp.int32),
        ],
        compiler_params=pltpu.CompilerParams(
            fuse_transposed_lhs_in_matmul=True),
    )(img4, W1, b1.reshape(1, COUT), g1.reshape(1, COUT),
      be1.reshape(1, COUT), cbext)


def _gather_rows(table, idx):
    mesh = plsc.VectorSubcoreMesh(core_axis_name="c", subcore_axis_name="s")
    n = idx.shape[0]
    tok_per_w = n // SC_NW

    @functools.partial(
        pl.kernel, mesh=mesh,
        out_type=jax.ShapeDtypeStruct((n, 2 * COUT), jnp.float32),
        scratch_types=[
            pltpu.VMEM((tok_per_w,), jnp.int32),
            pltpu.VMEM((tok_per_w, 2 * COUT), jnp.float32),
            pltpu.SemaphoreType.DMA,
        ],
    )
    def k(table_hbm, idx_hbm, out_hbm, idx_v, rows_v, sem):
        wid = lax.axis_index("s") * SC_NC + lax.axis_index("c")
        base = wid * tok_per_w
        pltpu.sync_copy(idx_hbm.at[pl.ds(base, tok_per_w)], idx_v)
        pltpu.async_copy(table_hbm.at[idx_v], rows_v, sem).wait()
        pltpu.sync_copy(rows_v, out_hbm.at[pl.ds(base, tok_per_w)])

    return k(table, idx)


def _finish_body(q_ref, h_ref, idx_ref, bern_ref, pos_ref, me_ref,
                 g2_ref, be2_ref, xqo_ref, lab_ref):
    idxv = idx_ref[0]
    bern = bern_ref[0]
    ti = lax.broadcasted_iota(jnp.int32, (T, 1), 0)
    tl = jnp.sum(jnp.where(ti == _TMP, idxv, 0), axis=0, keepdims=True)
    msk = (idxv == tl) & (bern > 0.5)
    mf = msk.astype(jnp.float32)
    emb = q_ref[0][:, :COUT] + h_ref[0]
    emb = emb * (1.0 - mf) + me_ref[0] * mf
    emb = emb + pos_ref[...]
    m = jnp.mean(emb, axis=-1, keepdims=True)
    v = jnp.mean((emb - m) ** 2, axis=-1, keepdims=True)
    xqo_ref[0] = (emb - m) * lax.rsqrt(v + 1e-5) * g2_ref[0] + be2_ref[0]
    lab_ref[0] = jnp.where(msk, idxv, -100)


def _finish(q3, h3, idx3, bern3, mask_emb, g2, be2):
    nb = q3.shape[0]
    return pl.pallas_call(
        _finish_body,
        grid=(nb,),
        in_specs=[
            pl.BlockSpec((1, T, 2 * COUT), lambda b: (b, 0, 0)),
            pl.BlockSpec((1, T, COUT), lambda b: (b, 0, 0)),
            pl.BlockSpec((1, T, 1), lambda b: (b, 0, 0)),
            pl.BlockSpec((1, T, 1), lambda b: (b, 0, 0)),
            pl.BlockSpec((T, COUT), lambda b: (0, 0)),
            pl.BlockSpec((1, COUT), lambda b: (0, 0)),
            pl.BlockSpec((1, COUT), lambda b: (0, 0)),
            pl.BlockSpec((1, COUT), lambda b: (0, 0)),
        ],
        out_specs=[
            pl.BlockSpec((1, T, COUT), lambda b: (b, 0, 0)),
            pl.BlockSpec((1, T, 1), lambda b: (b, 0, 0)),
        ],
        out_shape=[
            jax.ShapeDtypeStruct((nb, T, COUT), jnp.float32),
            jax.ShapeDtypeStruct((nb, T, 1), jnp.int32),
        ],
    )(q3, h3, idx3, bern3, jnp.asarray(_POS), mask_emb,
      g2.reshape(1, COUT), be2.reshape(1, COUT))


def kernel(img, W1, b1, g1, be1, codebook, Wp, bp, mask_emb, g2, be2):
    img4 = img.reshape(B, CIN, SY, 128)

    tbl, cbext = _codebook_xform(codebook, Wp, bp)

    bern = jax.random.bernoulli(jax.random.key(42), MASK_PROB,
                                (B, 1, 1)).astype(jnp.float32)
    bern3 = jnp.broadcast_to(bern, (B, T, 1))

    h4, idx3d = _encode_vq(img4, W1, b1, g1, be1, cbext)
    q = _gather_rows(tbl, idx3d.reshape(NTOK))
    xqo, lab = _finish(q.reshape(B, T, 2 * COUT), h4.reshape(B, T, COUT),
                       idx3d, bern3, mask_emb, g2, be2)
    vm = jnp.ones((B, T), jnp.int32)
    return (xqo, vm, lab.reshape(B, T))


# speedup vs baseline: 1.2820x; 1.0927x over previous
import functools
import math

import jax
import jax.numpy as jnp
import numpy as np
from jax import lax
from jax.experimental import pallas as pl
from jax.experimental.pallas import tpu as pltpu
from jax.experimental.pallas import tpu_sc as plsc

B = 16
CIN = 768
COUT = 64
K = 8192
H = 16
W = 16
T = 256
SY = 8
SX = 32
NTOK = B * T
CT = CIN
KC = 2048
KT = 1024
MASK_PROB = 0.015

SC_NC = 2
SC_NS = 16
SC_NW = SC_NC * SC_NS
TOK_PER_W = NTOK // SC_NW


def _pos_tokens() -> np.ndarray:
    mask = np.ones((1, H, W), np.float64)
    y_embed = np.cumsum(mask, axis=1)
    x_embed = np.cumsum(mask, axis=2)
    eps = 1e-6
    y_embed = y_embed / (y_embed[:, -1:, :] + eps) * 2 * math.pi
    x_embed = x_embed / (x_embed[:, :, -1:] + eps) * 2 * math.pi
    d = COUT // 2
    dim_t = np.arange(d, dtype=np.float64)
    dim_t = 10000.0 ** (2 * np.floor(dim_t / 2) / d)
    px = x_embed[:, :, :, None] / dim_t
    py = y_embed[:, :, :, None] / dim_t
    px = np.stack((np.sin(px[:, :, :, 0::2]), np.cos(px[:, :, :, 1::2])),
                  axis=4).reshape(1, H, W, -1)
    py = np.stack((np.sin(py[:, :, :, 0::2]), np.cos(py[:, :, :, 1::2])),
                  axis=4).reshape(1, H, W, -1)
    pos = np.concatenate((py, px), axis=3)
    return pos.reshape(T, COUT).astype(np.float32)


_POS = _pos_tokens()
np.random.seed(0)
_TMP = int(np.random.randint(H * W))


def _codebook_xform_body(cb_ref, wp_ref, bp_ref, t_ref, cbe_ref):
    cb = cb_ref[...]
    t = (jnp.dot(cb, wp_ref[...],
                 preferred_element_type=jnp.float32) + bp_ref[0])
    t_ref[...] = jnp.concatenate(
        [t, jnp.zeros((KT, COUT), jnp.float32)], axis=-1)
    cn = jnp.sum(cb * cb, axis=-1, keepdims=True)
    cbe_ref[...] = jnp.concatenate([-2.0 * cb, cn], axis=-1)


def _codebook_xform(codebook, Wp, bp):
    return pl.pallas_call(
        _codebook_xform_body,
        grid=(K // KT,),
        in_specs=[
            pl.BlockSpec((KT, COUT), lambda k: (k, 0)),
            pl.BlockSpec((COUT, COUT), lambda k: (0, 0)),
            pl.BlockSpec((1, COUT), lambda k: (0, 0)),
        ],
        out_specs=[
            pl.BlockSpec((KT, 2 * COUT), lambda k: (k, 0)),
            pl.BlockSpec((KT, COUT + 1), lambda k: (k, 0)),
        ],
        out_shape=[
            jax.ShapeDtypeStruct((K, 2 * COUT), jnp.float32),
            jax.ShapeDtypeStruct((K, COUT + 1), jnp.float32),
        ],
    )(codebook, Wp, bp.reshape(1, COUT))


def _encode_vq_body(img_ref, w1_ref, b1_ref, g1_ref, be1_ref, cbe_ref,
                    h_ref, idx_ref):
    x = img_ref[0]
    m1 = jnp.maximum(x, pltpu.roll(x, 96, axis=2))
    m2 = jnp.maximum(m1, pltpu.roll(m1, 127, axis=2))
    li = lax.broadcasted_iota(jnp.int32, (128, 32), 0)
    ji = lax.broadcasted_iota(jnp.int32, (128, 32), 1)
    sel = (li == 2 * ji + 32 * (ji // 16)).astype(jnp.float32)
    p = lax.dot_general(m2, sel, (((2,), (0,)), ((), ())),
                        preferred_element_type=jnp.float32)
    part = lax.dot_general(p, w1_ref[...], (((0,), (0,)), ((), ())),
                           preferred_element_type=jnp.float32)

    h1 = part + b1_ref[0]
    m = jnp.mean(h1, axis=-1, keepdims=True)
    v = jnp.mean((h1 - m) ** 2, axis=-1, keepdims=True)
    h1 = (h1 - m) * lax.rsqrt(v + 1e-5) * g1_ref[0] + be1_ref[0]
    h1 = jnp.maximum(h1, 0.0)
    h_ref[0] = h1
    hp = jnp.concatenate(
        [h1.reshape(T, COUT), jnp.ones((T, 1), jnp.float32)],
        axis=-1)
    li = lax.broadcasted_iota(jnp.int32, (T, KC), 1)

    def vq_step(kc, carry):
        bv, bi = carry
        off = pl.multiple_of(kc * KC, KC)
        cb = cbe_ref[pl.ds(off, KC), :]
        s = lax.dot_general(hp, cb, (((1,), (1,)), ((), ())),
                            preferred_element_type=jnp.float32)
        mv = jnp.min(s, axis=-1, keepdims=True)
        ai = jnp.min(jnp.where(s == mv, li, KC), axis=-1,
                     keepdims=True)
        gi = ai + kc * KC
        upd = mv < bv
        return jnp.where(upd, mv, bv), jnp.where(upd, gi, bi)

    bv0 = jnp.full((T, 1), jnp.inf, jnp.float32)
    bi0 = jnp.zeros((T, 1), jnp.int32)
    _, bi = lax.fori_loop(0, K // KC, vq_step, (bv0, bi0))
    idx_ref[0] = bi


def _encode_vq(img4, W1, b1, g1, be1, cbext):
    nb = img4.shape[0]
    return pl.pallas_call(
        _encode_vq_body,
        grid=(nb,),
        in_specs=[
            pl.BlockSpec((1, CT, SY, 128), lambda b: (b, 0, 0, 0)),
            pl.BlockSpec((CT, COUT), lambda b: (0, 0)),
            pl.BlockSpec((1, COUT), lambda b: (0, 0)),
            pl.BlockSpec((1, COUT), lambda b: (0, 0)),
            pl.BlockSpec((1, COUT), lambda b: (0, 0)),
            pl.BlockSpec((K, COUT + 1), lambda b: (0, 0)),
        ],
        out_specs=[
            pl.BlockSpec((1, SY, SX, COUT), lambda b: (b, 0, 0, 0)),
            pl.BlockSpec((1, T, 1), lambda b: (b, 0, 0)),
        ],
        out_shape=[
            jax.ShapeDtypeStruct((nb, SY, SX, COUT), jnp.float32),
            jax.ShapeDtypeStruct((nb, T, 1), jnp.int32),
        ],
        compiler_params=pltpu.CompilerParams(
            fuse_transposed_lhs_in_matmul=True),
    )(img4, W1, b1.reshape(1, COUT), g1.reshape(1, COUT),
      be1.reshape(1, COUT), cbext)


def _gather_rows(table, idx):
    mesh = plsc.VectorSubcoreMesh(core_axis_name="c", subcore_axis_name="s")
    n = idx.shape[0]
    tok_per_w = n // SC_NW

    @functools.partial(
        pl.kernel, mesh=mesh,
        out_type=jax.ShapeDtypeStruct((n, 2 * COUT), jnp.float32),
        scratch_types=[
            pltpu.VMEM((tok_per_w,), jnp.int32),
            pltpu.VMEM((tok_per_w, 2 * COUT), jnp.float32),
            pltpu.SemaphoreType.DMA,
        ],
    )
    def k(table_hbm, idx_hbm, out_hbm, idx_v, rows_v, sem):
        wid = lax.axis_index("s") * SC_NC + lax.axis_index("c")
        base = wid * tok_per_w
        pltpu.sync_copy(idx_hbm.at[pl.ds(base, tok_per_w)], idx_v)
        pltpu.async_copy(table_hbm.at[idx_v], rows_v, sem).wait()
        pltpu.sync_copy(rows_v, out_hbm.at[pl.ds(base, tok_per_w)])

    return k(table, idx)


def _finish_body(q_ref, h_ref, idx_ref, bern_ref, pos_ref, me_ref,
                 g2_ref, be2_ref, xqo_ref, lab_ref):
    idxv = idx_ref[0]
    bern = bern_ref[0]
    ti = lax.broadcasted_iota(jnp.int32, (T, 1), 0)
    tl = jnp.sum(jnp.where(ti == _TMP, idxv, 0), axis=0, keepdims=True)
    msk = (idxv == tl) & (bern > 0.5)
    mf = msk.astype(jnp.float32)
    emb = q_ref[0][:, :COUT] + h_ref[0]
    emb = emb * (1.0 - mf) + me_ref[0] * mf
    emb = emb + pos_ref[...]
    m = jnp.mean(emb, axis=-1, keepdims=True)
    v = jnp.mean((emb - m) ** 2, axis=-1, keepdims=True)
    xqo_ref[0] = (emb - m) * lax.rsqrt(v + 1e-5) * g2_ref[0] + be2_ref[0]
    lab_ref[0] = jnp.where(msk, idxv, -100)


def _finish(q3, h3, idx3, bern3, mask_emb, g2, be2):
    nb = q3.shape[0]
    return pl.pallas_call(
        _finish_body,
        grid=(nb,),
        in_specs=[
            pl.BlockSpec((1, T, 2 * COUT), lambda b: (b, 0, 0)),
            pl.BlockSpec((1, T, COUT), lambda b: (b, 0, 0)),
            pl.BlockSpec((1, T, 1), lambda b: (b, 0, 0)),
            pl.BlockSpec((1, T, 1), lambda b: (b, 0, 0)),
            pl.BlockSpec((T, COUT), lambda b: (0, 0)),
            pl.BlockSpec((1, COUT), lambda b: (0, 0)),
            pl.BlockSpec((1, COUT), lambda b: (0, 0)),
            pl.BlockSpec((1, COUT), lambda b: (0, 0)),
        ],
        out_specs=[
            pl.BlockSpec((1, T, COUT), lambda b: (b, 0, 0)),
            pl.BlockSpec((1, T, 1), lambda b: (b, 0, 0)),
        ],
        out_shape=[
            jax.ShapeDtypeStruct((nb, T, COUT), jnp.float32),
            jax.ShapeDtypeStruct((nb, T, 1), jnp.int32),
        ],
    )(q3, h3, idx3, bern3, jnp.asarray(_POS), mask_emb,
      g2.reshape(1, COUT), be2.reshape(1, COUT))


def kernel(img, W1, b1, g1, be1, codebook, Wp, bp, mask_emb, g2, be2):
    img4 = img.reshape(B, CIN, SY, 128)

    tbl, cbext = _codebook_xform(codebook, Wp, bp)

    bern = jax.random.bernoulli(jax.random.key(42), MASK_PROB,
                                (B, 1, 1)).astype(jnp.float32)
    bern3 = jnp.broadcast_to(bern, (B, T, 1))

    h4, idx3d = _encode_vq(img4, W1, b1, g1, be1, cbext)
    q = _gather_rows(tbl, idx3d.reshape(NTOK))
    xqo, lab = _finish(q.reshape(B, T, 2 * COUT), h4.reshape(B, T, COUT),
                       idx3d, bern3, mask_emb, g2, be2)
    vm = jnp.ones((B, T), jnp.int32)
    return (xqo, vm, lab.reshape(B, T))


# speedup vs baseline: 1.3495x; 1.0527x over previous
import functools
import math

import jax
import jax.numpy as jnp
import numpy as np
from jax import lax
from jax.experimental import pallas as pl
from jax.experimental.pallas import tpu as pltpu
from jax.experimental.pallas import tpu_sc as plsc

B = 16
CIN = 768
COUT = 64
K = 8192
H = 16
W = 16
T = 256
SY = 8
SX = 32
NTOK = B * T
CT = CIN
KC = 4096
KT = 1024
MASK_PROB = 0.015

SC_NC = 2
SC_NS = 16
SC_NW = SC_NC * SC_NS
TOK_PER_W = NTOK // SC_NW


def _pos_tokens() -> np.ndarray:
    mask = np.ones((1, H, W), np.float64)
    y_embed = np.cumsum(mask, axis=1)
    x_embed = np.cumsum(mask, axis=2)
    eps = 1e-6
    y_embed = y_embed / (y_embed[:, -1:, :] + eps) * 2 * math.pi
    x_embed = x_embed / (x_embed[:, :, -1:] + eps) * 2 * math.pi
    d = COUT // 2
    dim_t = np.arange(d, dtype=np.float64)
    dim_t = 10000.0 ** (2 * np.floor(dim_t / 2) / d)
    px = x_embed[:, :, :, None] / dim_t
    py = y_embed[:, :, :, None] / dim_t
    px = np.stack((np.sin(px[:, :, :, 0::2]), np.cos(px[:, :, :, 1::2])),
                  axis=4).reshape(1, H, W, -1)
    py = np.stack((np.sin(py[:, :, :, 0::2]), np.cos(py[:, :, :, 1::2])),
                  axis=4).reshape(1, H, W, -1)
    pos = np.concatenate((py, px), axis=3)
    return pos.reshape(T, COUT).astype(np.float32)


_POS = _pos_tokens()
np.random.seed(0)
_TMP = int(np.random.randint(H * W))


def _codebook_xform_body(cb_ref, wp_ref, bp_ref, t_ref, cbe_ref):
    cb = cb_ref[...]
    t = (jnp.dot(cb, wp_ref[...],
                 preferred_element_type=jnp.float32) + bp_ref[0])
    t_ref[...] = jnp.concatenate(
        [t, jnp.zeros((KT, COUT), jnp.float32)], axis=-1)
    cn = jnp.sum(cb * cb, axis=-1, keepdims=True)
    cbe_ref[...] = jnp.concatenate([-2.0 * cb, cn], axis=-1)


def _codebook_xform(codebook, Wp, bp):
    return pl.pallas_call(
        _codebook_xform_body,
        grid=(K // KT,),
        in_specs=[
            pl.BlockSpec((KT, COUT), lambda k: (k, 0)),
            pl.BlockSpec((COUT, COUT), lambda k: (0, 0)),
            pl.BlockSpec((1, COUT), lambda k: (0, 0)),
        ],
        out_specs=[
            pl.BlockSpec((KT, 2 * COUT), lambda k: (k, 0)),
            pl.BlockSpec((KT, COUT + 1), lambda k: (k, 0)),
        ],
        out_shape=[
            jax.ShapeDtypeStruct((K, 2 * COUT), jnp.float32),
            jax.ShapeDtypeStruct((K, COUT + 1), jnp.float32),
        ],
    )(codebook, Wp, bp.reshape(1, COUT))


def _encode_vq_body(img_ref, w1_ref, b1_ref, g1_ref, be1_ref, cbe_ref,
                    h_ref, idx_ref):
    x = img_ref[0]
    m1 = jnp.maximum(x, pltpu.roll(x, 96, axis=2))
    m2 = jnp.maximum(m1, pltpu.roll(m1, 127, axis=2))
    li = lax.broadcasted_iota(jnp.int32, (128, 32), 0)
    ji = lax.broadcasted_iota(jnp.int32, (128, 32), 1)
    sel = (li == 2 * ji + 32 * (ji // 16)).astype(jnp.float32)
    p = lax.dot_general(m2, sel, (((2,), (0,)), ((), ())),
                        preferred_element_type=jnp.float32)
    part = lax.dot_general(p, w1_ref[...], (((0,), (0,)), ((), ())),
                           preferred_element_type=jnp.float32)

    h1 = part + b1_ref[0]
    m = jnp.mean(h1, axis=-1, keepdims=True)
    v = jnp.mean((h1 - m) ** 2, axis=-1, keepdims=True)
    h1 = (h1 - m) * lax.rsqrt(v + 1e-5) * g1_ref[0] + be1_ref[0]
    h1 = jnp.maximum(h1, 0.0)
    h_ref[0] = h1
    hp = jnp.concatenate(
        [h1.reshape(T, COUT), jnp.ones((T, 1), jnp.float32)],
        axis=-1)
    li = lax.broadcasted_iota(jnp.int32, (T, KC), 1)

    def vq_step(kc, carry):
        bv, bi = carry
        off = pl.multiple_of(kc * KC, KC)
        cb = cbe_ref[pl.ds(off, KC), :]
        s = lax.dot_general(hp, cb, (((1,), (1,)), ((), ())),
                            preferred_element_type=jnp.float32)
        mv = jnp.min(s, axis=-1, keepdims=True)
        ai = jnp.min(jnp.where(s == mv, li, KC), axis=-1,
                     keepdims=True)
        gi = ai + kc * KC
        upd = mv < bv
        return jnp.where(upd, mv, bv), jnp.where(upd, gi, bi)

    bv0 = jnp.full((T, 1), jnp.inf, jnp.float32)
    bi0 = jnp.zeros((T, 1), jnp.int32)
    _, bi = lax.fori_loop(0, K // KC, vq_step, (bv0, bi0))
    idx_ref[0] = bi


def _encode_vq(img4, W1, b1, g1, be1, cbext):
    nb = img4.shape[0]
    return pl.pallas_call(
        _encode_vq_body,
        grid=(nb,),
        in_specs=[
            pl.BlockSpec((1, CT, SY, 128), lambda b: (b, 0, 0, 0)),
            pl.BlockSpec((CT, COUT), lambda b: (0, 0)),
            pl.BlockSpec((1, COUT), lambda b: (0, 0)),
            pl.BlockSpec((1, COUT), lambda b: (0, 0)),
            pl.BlockSpec((1, COUT), lambda b: (0, 0)),
            pl.BlockSpec((K, COUT + 1), lambda b: (0, 0)),
        ],
        out_specs=[
            pl.BlockSpec((1, SY, SX, COUT), lambda b: (b, 0, 0, 0)),
            pl.BlockSpec((1, T, 1), lambda b: (b, 0, 0)),
        ],
        out_shape=[
            jax.ShapeDtypeStruct((nb, SY, SX, COUT), jnp.float32),
            jax.ShapeDtypeStruct((nb, T, 1), jnp.int32),
        ],
        compiler_params=pltpu.CompilerParams(
            fuse_transposed_lhs_in_matmul=True),
    )(img4, W1, b1.reshape(1, COUT), g1.reshape(1, COUT),
      be1.reshape(1, COUT), cbext)


def _gather_rows(table, idx):
    mesh = plsc.VectorSubcoreMesh(core_axis_name="c", subcore_axis_name="s")
    n = idx.shape[0]
    tok_per_w = n // SC_NW

    @functools.partial(
        pl.kernel, mesh=mesh,
        out_type=jax.ShapeDtypeStruct((n, 2 * COUT), jnp.float32),
        scratch_types=[
            pltpu.VMEM((tok_per_w,), jnp.int32),
            pltpu.VMEM((tok_per_w, 2 * COUT), jnp.float32),
            pltpu.SemaphoreType.DMA,
        ],
    )
    def k(table_hbm, idx_hbm, out_hbm, idx_v, rows_v, sem):
        wid = lax.axis_index("s") * SC_NC + lax.axis_index("c")
        base = wid * tok_per_w
        pltpu.sync_copy(idx_hbm.at[pl.ds(base, tok_per_w)], idx_v)
        pltpu.async_copy(table_hbm.at[idx_v], rows_v, sem).wait()
        pltpu.sync_copy(rows_v, out_hbm.at[pl.ds(base, tok_per_w)])

    return k(table, idx)


def _finish_body(q_ref, h_ref, idx_ref, bern_ref, pos_ref, me_ref,
                 g2_ref, be2_ref, xqo_ref, lab_ref):
    idxv = idx_ref[0]
    bern = bern_ref[0]
    ti = lax.broadcasted_iota(jnp.int32, (T, 1), 0)
    tl = jnp.sum(jnp.where(ti == _TMP, idxv, 0), axis=0, keepdims=True)
    msk = (idxv == tl) & (bern > 0.5)
    mf = msk.astype(jnp.float32)
    emb = q_ref[0][:, :COUT] + h_ref[0]
    emb = emb * (1.0 - mf) + me_ref[0] * mf
    emb = emb + pos_ref[...]
    m = jnp.mean(emb, axis=-1, keepdims=True)
    v = jnp.mean((emb - m) ** 2, axis=-1, keepdims=True)
    xqo_ref[0] = (emb - m) * lax.rsqrt(v + 1e-5) * g2_ref[0] + be2_ref[0]
    lab_ref[0] = jnp.where(msk, idxv, -100)


def _finish(q3, h3, idx3, bern3, mask_emb, g2, be2):
    nb = q3.shape[0]
    return pl.pallas_call(
        _finish_body,
        grid=(nb,),
        in_specs=[
            pl.BlockSpec((1, T, 2 * COUT), lambda b: (b, 0, 0)),
            pl.BlockSpec((1, T, COUT), lambda b: (b, 0, 0)),
            pl.BlockSpec((1, T, 1), lambda b: (b, 0, 0)),
            pl.BlockSpec((1, T, 1), lambda b: (b, 0, 0)),
            pl.BlockSpec((T, COUT), lambda b: (0, 0)),
            pl.BlockSpec((1, COUT), lambda b: (0, 0)),
            pl.BlockSpec((1, COUT), lambda b: (0, 0)),
            pl.BlockSpec((1, COUT), lambda b: (0, 0)),
        ],
        out_specs=[
            pl.BlockSpec((1, T, COUT), lambda b: (b, 0, 0)),
            pl.BlockSpec((1, T, 1), lambda b: (b, 0, 0)),
        ],
        out_shape=[
            jax.ShapeDtypeStruct((nb, T, COUT), jnp.float32),
            jax.ShapeDtypeStruct((nb, T, 1), jnp.int32),
        ],
    )(q3, h3, idx3, bern3, jnp.asarray(_POS), mask_emb,
      g2.reshape(1, COUT), be2.reshape(1, COUT))


def kernel(img, W1, b1, g1, be1, codebook, Wp, bp, mask_emb, g2, be2):
    img4 = img.reshape(B, CIN, SY, 128)

    tbl, cbext = _codebook_xform(codebook, Wp, bp)

    bern = jax.random.bernoulli(jax.random.key(42), MASK_PROB,
                                (B, 1, 1)).astype(jnp.float32)
    bern3 = jnp.broadcast_to(bern, (B, T, 1))

    h4, idx3d = _encode_vq(img4, W1, b1, g1, be1, cbext)
    q = _gather_rows(tbl, idx3d.reshape(NTOK))
    xqo, lab = _finish(q.reshape(B, T, 2 * COUT), h4.reshape(B, T, COUT),
                       idx3d, bern3, mask_emb, g2, be2)
    vm = jnp.ones((B, T), jnp.int32)
    return (xqo, vm, lab.reshape(B, T))


# speedup vs baseline: 1.4200x; 1.0522x over previous
import functools
import math

import jax
import jax.numpy as jnp
import numpy as np
from jax import lax
from jax.experimental import pallas as pl
from jax.experimental.pallas import tpu as pltpu
from jax.experimental.pallas import tpu_sc as plsc

B = 16
CIN = 768
COUT = 64
K = 8192
H = 16
W = 16
T = 256
SY = 8
SX = 32
NTOK = B * T
CT = CIN
KC = 8192
KT = 1024
MASK_PROB = 0.015

SC_NC = 2
SC_NS = 16
SC_NW = SC_NC * SC_NS
TOK_PER_W = NTOK // SC_NW


def _pos_tokens() -> np.ndarray:
    mask = np.ones((1, H, W), np.float64)
    y_embed = np.cumsum(mask, axis=1)
    x_embed = np.cumsum(mask, axis=2)
    eps = 1e-6
    y_embed = y_embed / (y_embed[:, -1:, :] + eps) * 2 * math.pi
    x_embed = x_embed / (x_embed[:, :, -1:] + eps) * 2 * math.pi
    d = COUT // 2
    dim_t = np.arange(d, dtype=np.float64)
    dim_t = 10000.0 ** (2 * np.floor(dim_t / 2) / d)
    px = x_embed[:, :, :, None] / dim_t
    py = y_embed[:, :, :, None] / dim_t
    px = np.stack((np.sin(px[:, :, :, 0::2]), np.cos(px[:, :, :, 1::2])),
                  axis=4).reshape(1, H, W, -1)
    py = np.stack((np.sin(py[:, :, :, 0::2]), np.cos(py[:, :, :, 1::2])),
                  axis=4).reshape(1, H, W, -1)
    pos = np.concatenate((py, px), axis=3)
    return pos.reshape(T, COUT).astype(np.float32)


_POS = _pos_tokens()
np.random.seed(0)
_TMP = int(np.random.randint(H * W))


def _codebook_xform_body(cb_ref, wp_ref, bp_ref, t_ref, cbe_ref):
    cb = cb_ref[...]
    t = (jnp.dot(cb, wp_ref[...],
                 preferred_element_type=jnp.float32) + bp_ref[0])
    t_ref[...] = jnp.concatenate(
        [t, jnp.zeros((KT, COUT), jnp.float32)], axis=-1)
    cn = jnp.sum(cb * cb, axis=-1, keepdims=True)
    cbe_ref[...] = jnp.concatenate([-2.0 * cb, cn], axis=-1)


def _codebook_xform(codebook, Wp, bp):
    return pl.pallas_call(
        _codebook_xform_body,
        grid=(K // KT,),
        in_specs=[
            pl.BlockSpec((KT, COUT), lambda k: (k, 0)),
            pl.BlockSpec((COUT, COUT), lambda k: (0, 0)),
            pl.BlockSpec((1, COUT), lambda k: (0, 0)),
        ],
        out_specs=[
            pl.BlockSpec((KT, 2 * COUT), lambda k: (k, 0)),
            pl.BlockSpec((KT, COUT + 1), lambda k: (k, 0)),
        ],
        out_shape=[
            jax.ShapeDtypeStruct((K, 2 * COUT), jnp.float32),
            jax.ShapeDtypeStruct((K, COUT + 1), jnp.float32),
        ],
    )(codebook, Wp, bp.reshape(1, COUT))


def _encode_vq_body(img_ref, w1_ref, b1_ref, g1_ref, be1_ref, cbe_ref,
                    h_ref, idx_ref):
    x = img_ref[0]
    m1 = jnp.maximum(x, pltpu.roll(x, 96, axis=2))
    m2 = jnp.maximum(m1, pltpu.roll(m1, 127, axis=2))
    li = lax.broadcasted_iota(jnp.int32, (128, 32), 0)
    ji = lax.broadcasted_iota(jnp.int32, (128, 32), 1)
    sel = (li == 2 * ji + 32 * (ji // 16)).astype(jnp.float32)
    p = lax.dot_general(m2, sel, (((2,), (0,)), ((), ())),
                        preferred_element_type=jnp.float32)
    part = lax.dot_general(p, w1_ref[...], (((0,), (0,)), ((), ())),
                           preferred_element_type=jnp.float32)

    h1 = part + b1_ref[0]
    m = jnp.mean(h1, axis=-1, keepdims=True)
    v = jnp.mean((h1 - m) ** 2, axis=-1, keepdims=True)
    h1 = (h1 - m) * lax.rsqrt(v + 1e-5) * g1_ref[0] + be1_ref[0]
    h1 = jnp.maximum(h1, 0.0)
    h_ref[0] = h1
    hp = jnp.concatenate(
        [h1.reshape(T, COUT), jnp.ones((T, 1), jnp.float32)],
        axis=-1)
    li = lax.broadcasted_iota(jnp.int32, (T, KC), 1)

    def vq_step(kc, carry):
        bv, bi = carry
        off = pl.multiple_of(kc * KC, KC)
        cb = cbe_ref[pl.ds(off, KC), :]
        s = lax.dot_general(hp, cb, (((1,), (1,)), ((), ())),
                            preferred_element_type=jnp.float32)
        mv = jnp.min(s, axis=-1, keepdims=True)
        ai = jnp.min(jnp.where(s == mv, li, KC), axis=-1,
                     keepdims=True)
        gi = ai + kc * KC
        upd = mv < bv
        return jnp.where(upd, mv, bv), jnp.where(upd, gi, bi)

    bv0 = jnp.full((T, 1), jnp.inf, jnp.float32)
    bi0 = jnp.zeros((T, 1), jnp.int32)
    _, bi = lax.fori_loop(0, K // KC, vq_step, (bv0, bi0))
    idx_ref[0] = bi


def _encode_vq(img4, W1, b1, g1, be1, cbext):
    nb = img4.shape[0]
    return pl.pallas_call(
        _encode_vq_body,
        grid=(nb,),
        in_specs=[
            pl.BlockSpec((1, CT, SY, 128), lambda b: (b, 0, 0, 0)),
            pl.BlockSpec((CT, COUT), lambda b: (0, 0)),
            pl.BlockSpec((1, COUT), lambda b: (0, 0)),
            pl.BlockSpec((1, COUT), lambda b: (0, 0)),
            pl.BlockSpec((1, COUT), lambda b: (0, 0)),
            pl.BlockSpec((K, COUT + 1), lambda b: (0, 0)),
        ],
        out_specs=[
            pl.BlockSpec((1, SY, SX, COUT), lambda b: (b, 0, 0, 0)),
            pl.BlockSpec((1, T, 1), lambda b: (b, 0, 0)),
        ],
        out_shape=[
            jax.ShapeDtypeStruct((nb, SY, SX, COUT), jnp.float32),
            jax.ShapeDtypeStruct((nb, T, 1), jnp.int32),
        ],
        compiler_params=pltpu.CompilerParams(
            fuse_transposed_lhs_in_matmul=True),
    )(img4, W1, b1.reshape(1, COUT), g1.reshape(1, COUT),
      be1.reshape(1, COUT), cbext)


def _gather_rows(table, idx):
    mesh = plsc.VectorSubcoreMesh(core_axis_name="c", subcore_axis_name="s")
    n = idx.shape[0]
    tok_per_w = n // SC_NW

    @functools.partial(
        pl.kernel, mesh=mesh,
        out_type=jax.ShapeDtypeStruct((n, 2 * COUT), jnp.float32),
        scratch_types=[
            pltpu.VMEM((tok_per_w,), jnp.int32),
            pltpu.VMEM((tok_per_w, 2 * COUT), jnp.float32),
            pltpu.SemaphoreType.DMA,
        ],
    )
    def k(table_hbm, idx_hbm, out_hbm, idx_v, rows_v, sem):
        wid = lax.axis_index("s") * SC_NC + lax.axis_index("c")
        base = wid * tok_per_w
        pltpu.sync_copy(idx_hbm.at[pl.ds(base, tok_per_w)], idx_v)
        pltpu.async_copy(table_hbm.at[idx_v], rows_v, sem).wait()
        pltpu.sync_copy(rows_v, out_hbm.at[pl.ds(base, tok_per_w)])

    return k(table, idx)


def _finish_body(q_ref, h_ref, idx_ref, bern_ref, pos_ref, me_ref,
                 g2_ref, be2_ref, xqo_ref, lab_ref):
    idxv = idx_ref[0]
    bern = bern_ref[0]
    ti = lax.broadcasted_iota(jnp.int32, (T, 1), 0)
    tl = jnp.sum(jnp.where(ti == _TMP, idxv, 0), axis=0, keepdims=True)
    msk = (idxv == tl) & (bern > 0.5)
    mf = msk.astype(jnp.float32)
    emb = q_ref[0][:, :COUT] + h_ref[0]
    emb = emb * (1.0 - mf) + me_ref[0] * mf
    emb = emb + pos_ref[...]
    m = jnp.mean(emb, axis=-1, keepdims=True)
    v = jnp.mean((emb - m) ** 2, axis=-1, keepdims=True)
    xqo_ref[0] = (emb - m) * lax.rsqrt(v + 1e-5) * g2_ref[0] + be2_ref[0]
    lab_ref[0] = jnp.where(msk, idxv, -100)


def _finish(q3, h3, idx3, bern3, mask_emb, g2, be2):
    nb = q3.shape[0]
    return pl.pallas_call(
        _finish_body,
        grid=(nb,),
        in_specs=[
            pl.BlockSpec((1, T, 2 * COUT), lambda b: (b, 0, 0)),
            pl.BlockSpec((1, T, COUT), lambda b: (b, 0, 0)),
            pl.BlockSpec((1, T, 1), lambda b: (b, 0, 0)),
            pl.BlockSpec((1, T, 1), lambda b: (b, 0, 0)),
            pl.BlockSpec((T, COUT), lambda b: (0, 0)),
            pl.BlockSpec((1, COUT), lambda b: (0, 0)),
            pl.BlockSpec((1, COUT), lambda b: (0, 0)),
            pl.BlockSpec((1, COUT), lambda b: (0, 0)),
        ],
        out_specs=[
            pl.BlockSpec((1, T, COUT), lambda b: (b, 0, 0)),
            pl.BlockSpec((1, T, 1), lambda b: (b, 0, 0)),
        ],
        out_shape=[
            jax.ShapeDtypeStruct((nb, T, COUT), jnp.float32),
            jax.ShapeDtypeStruct((nb, T, 1), jnp.int32),
        ],
    )(q3, h3, idx3, bern3, jnp.asarray(_POS), mask_emb,
      g2.reshape(1, COUT), be2.reshape(1, COUT))


def kernel(img, W1, b1, g1, be1, codebook, Wp, bp, mask_emb, g2, be2):
    img4 = img.reshape(B, CIN, SY, 128)

    tbl, cbext = _codebook_xform(codebook, Wp, bp)

    bern = jax.random.bernoulli(jax.random.key(42), MASK_PROB,
                                (B, 1, 1)).astype(jnp.float32)
    bern3 = jnp.broadcast_to(bern, (B, T, 1))

    h4, idx3d = _encode_vq(img4, W1, b1, g1, be1, cbext)
    q = _gather_rows(tbl, idx3d.reshape(NTOK))
    xqo, lab = _finish(q.reshape(B, T, 2 * COUT), h4.reshape(B, T, COUT),
                       idx3d, bern3, mask_emb, g2, be2)
    vm = jnp.ones((B, T), jnp.int32)
    return (xqo, vm, lab.reshape(B, T))


# speedup vs baseline: 1.5368x; 1.0823x over previous
import functools
import math

import jax
import jax.numpy as jnp
import numpy as np
from jax import lax
from jax.experimental import pallas as pl
from jax.experimental.pallas import tpu as pltpu
from jax.experimental.pallas import tpu_sc as plsc

B = 16
CIN = 768
COUT = 64
K = 8192
H = 16
W = 16
T = 256
SY = 8
SX = 32
NTOK = B * T
CT = CIN
KC = 8192
KT = 1024
MASK_PROB = 0.015

SC_NC = 2
SC_NS = 16
SC_NW = SC_NC * SC_NS
TOK_PER_W = NTOK // SC_NW


def _pos_tokens() -> np.ndarray:
    mask = np.ones((1, H, W), np.float64)
    y_embed = np.cumsum(mask, axis=1)
    x_embed = np.cumsum(mask, axis=2)
    eps = 1e-6
    y_embed = y_embed / (y_embed[:, -1:, :] + eps) * 2 * math.pi
    x_embed = x_embed / (x_embed[:, :, -1:] + eps) * 2 * math.pi
    d = COUT // 2
    dim_t = np.arange(d, dtype=np.float64)
    dim_t = 10000.0 ** (2 * np.floor(dim_t / 2) / d)
    px = x_embed[:, :, :, None] / dim_t
    py = y_embed[:, :, :, None] / dim_t
    px = np.stack((np.sin(px[:, :, :, 0::2]), np.cos(px[:, :, :, 1::2])),
                  axis=4).reshape(1, H, W, -1)
    py = np.stack((np.sin(py[:, :, :, 0::2]), np.cos(py[:, :, :, 1::2])),
                  axis=4).reshape(1, H, W, -1)
    pos = np.concatenate((py, px), axis=3)
    return pos.reshape(T, COUT).astype(np.float32)


_POS = _pos_tokens()
np.random.seed(0)
_TMP = int(np.random.randint(H * W))


def _codebook_xform_body(cb_ref, wp_ref, bp_ref, t_ref, cbe_ref):
    cb = cb_ref[...]
    t = (jnp.dot(cb, wp_ref[...],
                 preferred_element_type=jnp.float32) + bp_ref[0])
    t_ref[...] = jnp.concatenate(
        [t, jnp.zeros((KT, COUT), jnp.float32)], axis=-1)
    cn = jnp.sum(cb * cb, axis=-1, keepdims=True)
    cbe_ref[...] = jnp.concatenate([-2.0 * cb, cn], axis=-1)


def _codebook_xform(codebook, Wp, bp):
    return pl.pallas_call(
        _codebook_xform_body,
        grid=(K // KT,),
        in_specs=[
            pl.BlockSpec((KT, COUT), lambda k: (k, 0)),
            pl.BlockSpec((COUT, COUT), lambda k: (0, 0)),
            pl.BlockSpec((1, COUT), lambda k: (0, 0)),
        ],
        out_specs=[
            pl.BlockSpec((KT, 2 * COUT), lambda k: (k, 0)),
            pl.BlockSpec((KT, COUT + 1), lambda k: (k, 0)),
        ],
        out_shape=[
            jax.ShapeDtypeStruct((K, 2 * COUT), jnp.float32),
            jax.ShapeDtypeStruct((K, COUT + 1), jnp.float32),
        ],
    )(codebook, Wp, bp.reshape(1, COUT))


def _encode_vq_body(img_ref, w1_ref, b1_ref, g1_ref, be1_ref, cbe_ref,
                    h_ref, idx_ref):
    x = img_ref[0]
    m1 = jnp.maximum(x, pltpu.roll(x, 96, axis=2))
    m2 = jnp.maximum(m1, pltpu.roll(m1, 127, axis=2))
    li = lax.broadcasted_iota(jnp.int32, (128, 32), 0)
    ji = lax.broadcasted_iota(jnp.int32, (128, 32), 1)
    sel = (li == 2 * ji + 32 * (ji // 16)).astype(jnp.float32)
    p = lax.dot_general(m2, sel, (((2,), (0,)), ((), ())),
                        preferred_element_type=jnp.float32)
    part = lax.dot_general(p, w1_ref[...], (((0,), (0,)), ((), ())),
                           preferred_element_type=jnp.float32)

    h1 = part + b1_ref[0]
    m = jnp.mean(h1, axis=-1, keepdims=True)
    v = jnp.mean((h1 - m) ** 2, axis=-1, keepdims=True)
    h1 = (h1 - m) * lax.rsqrt(v + 1e-5) * g1_ref[0] + be1_ref[0]
    h1 = jnp.maximum(h1, 0.0)
    h_ref[0] = h1
    hp = jnp.concatenate(
        [h1.reshape(T, COUT), jnp.ones((T, 1), jnp.float32)],
        axis=-1)
    s = lax.dot_general(hp, cbe_ref[...], (((1,), (1,)), ((), ())),
                        preferred_element_type=jnp.float32)
    idx_ref[0] = jnp.argmin(s, axis=-1).astype(jnp.int32)[:, None]


def _encode_vq(img4, W1, b1, g1, be1, cbext):
    nb = img4.shape[0]
    return pl.pallas_call(
        _encode_vq_body,
        grid=(nb,),
        in_specs=[
            pl.BlockSpec((1, CT, SY, 128), lambda b: (b, 0, 0, 0)),
            pl.BlockSpec((CT, COUT), lambda b: (0, 0)),
            pl.BlockSpec((1, COUT), lambda b: (0, 0)),
            pl.BlockSpec((1, COUT), lambda b: (0, 0)),
            pl.BlockSpec((1, COUT), lambda b: (0, 0)),
            pl.BlockSpec((K, COUT + 1), lambda b: (0, 0)),
        ],
        out_specs=[
            pl.BlockSpec((1, SY, SX, COUT), lambda b: (b, 0, 0, 0)),
            pl.BlockSpec((1, T, 1), lambda b: (b, 0, 0)),
        ],
        out_shape=[
            jax.ShapeDtypeStruct((nb, SY, SX, COUT), jnp.float32),
            jax.ShapeDtypeStruct((nb, T, 1), jnp.int32),
        ],
        compiler_params=pltpu.CompilerParams(
            fuse_transposed_lhs_in_matmul=True),
    )(img4, W1, b1.reshape(1, COUT), g1.reshape(1, COUT),
      be1.reshape(1, COUT), cbext)


def _gather_rows(table, idx):
    mesh = plsc.VectorSubcoreMesh(core_axis_name="c", subcore_axis_name="s")
    n = idx.shape[0]
    tok_per_w = n // SC_NW

    @functools.partial(
        pl.kernel, mesh=mesh,
        out_type=jax.ShapeDtypeStruct((n, 2 * COUT), jnp.float32),
        scratch_types=[
            pltpu.VMEM((tok_per_w,), jnp.int32),
            pltpu.VMEM((tok_per_w, 2 * COUT), jnp.float32),
            pltpu.SemaphoreType.DMA,
        ],
    )
    def k(table_hbm, idx_hbm, out_hbm, idx_v, rows_v, sem):
        wid = lax.axis_index("s") * SC_NC + lax.axis_index("c")
        base = wid * tok_per_w
        pltpu.sync_copy(idx_hbm.at[pl.ds(base, tok_per_w)], idx_v)
        pltpu.async_copy(table_hbm.at[idx_v], rows_v, sem).wait()
        pltpu.sync_copy(rows_v, out_hbm.at[pl.ds(base, tok_per_w)])

    return k(table, idx)


def _finish_body(q_ref, h_ref, idx_ref, bern_ref, pos_ref, me_ref,
                 g2_ref, be2_ref, xqo_ref, lab_ref):
    idxv = idx_ref[0]
    bern = bern_ref[0]
    ti = lax.broadcasted_iota(jnp.int32, (T, 1), 0)
    tl = jnp.sum(jnp.where(ti == _TMP, idxv, 0), axis=0, keepdims=True)
    msk = (idxv == tl) & (bern > 0.5)
    mf = msk.astype(jnp.float32)
    emb = q_ref[0][:, :COUT] + h_ref[0]
    emb = emb * (1.0 - mf) + me_ref[0] * mf
    emb = emb + pos_ref[...]
    m = jnp.mean(emb, axis=-1, keepdims=True)
    v = jnp.mean((emb - m) ** 2, axis=-1, keepdims=True)
    xqo_ref[0] = (emb - m) * lax.rsqrt(v + 1e-5) * g2_ref[0] + be2_ref[0]
    lab_ref[0] = jnp.where(msk, idxv, -100)


def _finish(q3, h3, idx3, bern3, mask_emb, g2, be2):
    nb = q3.shape[0]
    return pl.pallas_call(
        _finish_body,
        grid=(nb,),
        in_specs=[
            pl.BlockSpec((1, T, 2 * COUT), lambda b: (b, 0, 0)),
            pl.BlockSpec((1, T, COUT), lambda b: (b, 0, 0)),
            pl.BlockSpec((1, T, 1), lambda b: (b, 0, 0)),
            pl.BlockSpec((1, T, 1), lambda b: (b, 0, 0)),
            pl.BlockSpec((T, COUT), lambda b: (0, 0)),
            pl.BlockSpec((1, COUT), lambda b: (0, 0)),
            pl.BlockSpec((1, COUT), lambda b: (0, 0)),
            pl.BlockSpec((1, COUT), lambda b: (0, 0)),
        ],
        out_specs=[
            pl.BlockSpec((1, T, COUT), lambda b: (b, 0, 0)),
            pl.BlockSpec((1, T, 1), lambda b: (b, 0, 0)),
        ],
        out_shape=[
            jax.ShapeDtypeStruct((nb, T, COUT), jnp.float32),
            jax.ShapeDtypeStruct((nb, T, 1), jnp.int32),
        ],
    )(q3, h3, idx3, bern3, jnp.asarray(_POS), mask_emb,
      g2.reshape(1, COUT), be2.reshape(1, COUT))


def kernel(img, W1, b1, g1, be1, codebook, Wp, bp, mask_emb, g2, be2):
    img4 = img.reshape(B, CIN, SY, 128)

    tbl, cbext = _codebook_xform(codebook, Wp, bp)

    bern = jax.random.bernoulli(jax.random.key(42), MASK_PROB,
                                (B, 1, 1)).astype(jnp.float32)
    bern3 = jnp.broadcast_to(bern, (B, T, 1))

    h4, idx3d = _encode_vq(img4, W1, b1, g1, be1, cbext)
    q = _gather_rows(tbl, idx3d.reshape(NTOK))
    xqo, lab = _finish(q.reshape(B, T, 2 * COUT), h4.reshape(B, T, COUT),
                       idx3d, bern3, mask_emb, g2, be2)
    vm = jnp.ones((B, T), jnp.int32)
    return (xqo, vm, lab.reshape(B, T))


# speedup vs baseline: 1.5641x; 1.0177x over previous
import functools
import math

import jax
import jax.numpy as jnp
import numpy as np
from jax import lax
from jax.experimental import pallas as pl
from jax.experimental.pallas import tpu as pltpu
from jax.experimental.pallas import tpu_sc as plsc

B = 16
CIN = 768
COUT = 64
K = 8192
H = 16
W = 16
T = 256
SY = 8
SX = 32
NTOK = B * T
CT = CIN
KC = 8192
KT = 1024
MASK_PROB = 0.015

SC_NC = 2
SC_NS = 16
SC_NW = SC_NC * SC_NS
TOK_PER_W = NTOK // SC_NW


def _pos_tokens() -> np.ndarray:
    mask = np.ones((1, H, W), np.float64)
    y_embed = np.cumsum(mask, axis=1)
    x_embed = np.cumsum(mask, axis=2)
    eps = 1e-6
    y_embed = y_embed / (y_embed[:, -1:, :] + eps) * 2 * math.pi
    x_embed = x_embed / (x_embed[:, :, -1:] + eps) * 2 * math.pi
    d = COUT // 2
    dim_t = np.arange(d, dtype=np.float64)
    dim_t = 10000.0 ** (2 * np.floor(dim_t / 2) / d)
    px = x_embed[:, :, :, None] / dim_t
    py = y_embed[:, :, :, None] / dim_t
    px = np.stack((np.sin(px[:, :, :, 0::2]), np.cos(px[:, :, :, 1::2])),
                  axis=4).reshape(1, H, W, -1)
    py = np.stack((np.sin(py[:, :, :, 0::2]), np.cos(py[:, :, :, 1::2])),
                  axis=4).reshape(1, H, W, -1)
    pos = np.concatenate((py, px), axis=3)
    return pos.reshape(T, COUT).astype(np.float32)


_POS = _pos_tokens()
np.random.seed(0)
_TMP = int(np.random.randint(H * W))

_SEL = np.zeros((1024, T), np.float32)
_SEL[64 * (np.arange(T) // 16) + 2 * (np.arange(T) % 16), np.arange(T)] = 1.0


def _codebook_xform_body(cb_ref, wp_ref, bp_ref, t_ref, cbe_ref):
    cb = cb_ref[...]
    t = (jnp.dot(cb, wp_ref[...],
                 preferred_element_type=jnp.float32) + bp_ref[0])
    t_ref[...] = jnp.concatenate(
        [t, jnp.zeros((KT, COUT), jnp.float32)], axis=-1)
    cn = jnp.sum(cb * cb, axis=-1, keepdims=True)
    cbe_ref[...] = jnp.concatenate([-2.0 * cb, cn], axis=-1)


def _codebook_xform(codebook, Wp, bp):
    return pl.pallas_call(
        _codebook_xform_body,
        grid=(K // KT,),
        in_specs=[
            pl.BlockSpec((KT, COUT), lambda k: (k, 0)),
            pl.BlockSpec((COUT, COUT), lambda k: (0, 0)),
            pl.BlockSpec((1, COUT), lambda k: (0, 0)),
        ],
        out_specs=[
            pl.BlockSpec((KT, 2 * COUT), lambda k: (k, 0)),
            pl.BlockSpec((KT, COUT + 1), lambda k: (k, 0)),
        ],
        out_shape=[
            jax.ShapeDtypeStruct((K, 2 * COUT), jnp.float32),
            jax.ShapeDtypeStruct((K, COUT + 1), jnp.float32),
        ],
    )(codebook, Wp, bp.reshape(1, COUT))


def _encode_vq_body(img_ref, w1_ref, b1_ref, g1_ref, be1_ref, cbe_ref,
                    sel_ref, h_ref, idx_ref):
    x = img_ref[0]
    m1 = jnp.maximum(x, pltpu.roll(x, 992, axis=1))
    m2 = jnp.maximum(m1, pltpu.roll(m1, 1023, axis=1))
    p2 = lax.dot_general(m2, sel_ref[...], (((1,), (0,)), ((), ())),
                         preferred_element_type=jnp.float32)
    part = lax.dot_general(p2, w1_ref[...], (((0,), (0,)), ((), ())),
                           preferred_element_type=jnp.float32)

    h1 = part + b1_ref[0]
    m = jnp.mean(h1, axis=-1, keepdims=True)
    v = jnp.mean((h1 - m) ** 2, axis=-1, keepdims=True)
    h1 = (h1 - m) * lax.rsqrt(v + 1e-5) * g1_ref[0] + be1_ref[0]
    h1 = jnp.maximum(h1, 0.0)
    h_ref[0] = h1
    hp = jnp.concatenate(
        [h1, jnp.ones((T, 1), jnp.float32)], axis=-1)
    s = lax.dot_general(hp, cbe_ref[...], (((1,), (1,)), ((), ())),
                        preferred_element_type=jnp.float32)
    idx_ref[0] = jnp.argmin(s, axis=-1).astype(jnp.int32)[:, None]


def _encode_vq(img4, W1, b1, g1, be1, cbext):
    nb = img4.shape[0]
    return pl.pallas_call(
        _encode_vq_body,
        grid=(nb,),
        in_specs=[
            pl.BlockSpec((1, CT, 1024), lambda b: (b, 0, 0)),
            pl.BlockSpec((CT, COUT), lambda b: (0, 0)),
            pl.BlockSpec((1, COUT), lambda b: (0, 0)),
            pl.BlockSpec((1, COUT), lambda b: (0, 0)),
            pl.BlockSpec((1, COUT), lambda b: (0, 0)),
            pl.BlockSpec((K, COUT + 1), lambda b: (0, 0)),
            pl.BlockSpec((1024, T), lambda b: (0, 0)),
        ],
        out_specs=[
            pl.BlockSpec((1, T, COUT), lambda b: (b, 0, 0)),
            pl.BlockSpec((1, T, 1), lambda b: (b, 0, 0)),
        ],
        out_shape=[
            jax.ShapeDtypeStruct((nb, T, COUT), jnp.float32),
            jax.ShapeDtypeStruct((nb, T, 1), jnp.int32),
        ],
        compiler_params=pltpu.CompilerParams(
            fuse_transposed_lhs_in_matmul=True),
    )(img4, W1, b1.reshape(1, COUT), g1.reshape(1, COUT),
      be1.reshape(1, COUT), cbext, jnp.asarray(_SEL))


def _gather_rows(table, idx):
    mesh = plsc.VectorSubcoreMesh(core_axis_name="c", subcore_axis_name="s")
    n = idx.shape[0]
    tok_per_w = n // SC_NW

    @functools.partial(
        pl.kernel, mesh=mesh,
        out_type=jax.ShapeDtypeStruct((n, 2 * COUT), jnp.float32),
        scratch_types=[
            pltpu.VMEM((tok_per_w,), jnp.int32),
            pltpu.VMEM((tok_per_w, 2 * COUT), jnp.float32),
            pltpu.SemaphoreType.DMA,
        ],
    )
    def k(table_hbm, idx_hbm, out_hbm, idx_v, rows_v, sem):
        wid = lax.axis_index("s") * SC_NC + lax.axis_index("c")
        base = wid * tok_per_w
        pltpu.sync_copy(idx_hbm.at[pl.ds(base, tok_per_w)], idx_v)
        pltpu.async_copy(table_hbm.at[idx_v], rows_v, sem).wait()
        pltpu.sync_copy(rows_v, out_hbm.at[pl.ds(base, tok_per_w)])

    return k(table, idx)


def _finish_body(q_ref, h_ref, idx_ref, bern_ref, pos_ref, me_ref,
                 g2_ref, be2_ref, xqo_ref, lab_ref):
    idxv = idx_ref[0]
    bern = bern_ref[0]
    ti = lax.broadcasted_iota(jnp.int32, (T, 1), 0)
    tl = jnp.sum(jnp.where(ti == _TMP, idxv, 0), axis=0, keepdims=True)
    msk = (idxv == tl) & (bern > 0.5)
    mf = msk.astype(jnp.float32)
    emb = q_ref[0][:, :COUT] + h_ref[0]
    emb = emb * (1.0 - mf) + me_ref[0] * mf
    emb = emb + pos_ref[...]
    m = jnp.mean(emb, axis=-1, keepdims=True)
    v = jnp.mean((emb - m) ** 2, axis=-1, keepdims=True)
    xqo_ref[0] = (emb - m) * lax.rsqrt(v + 1e-5) * g2_ref[0] + be2_ref[0]
    lab_ref[0] = jnp.where(msk, idxv, -100)


def _finish(q3, h3, idx3, bern3, mask_emb, g2, be2):
    nb = q3.shape[0]
    return pl.pallas_call(
        _finish_body,
        grid=(nb,),
        in_specs=[
            pl.BlockSpec((1, T, 2 * COUT), lambda b: (b, 0, 0)),
            pl.BlockSpec((1, T, COUT), lambda b: (b, 0, 0)),
            pl.BlockSpec((1, T, 1), lambda b: (b, 0, 0)),
            pl.BlockSpec((1, T, 1), lambda b: (b, 0, 0)),
            pl.BlockSpec((T, COUT), lambda b: (0, 0)),
            pl.BlockSpec((1, COUT), lambda b: (0, 0)),
            pl.BlockSpec((1, COUT), lambda b: (0, 0)),
            pl.BlockSpec((1, COUT), lambda b: (0, 0)),
        ],
        out_specs=[
            pl.BlockSpec((1, T, COUT), lambda b: (b, 0, 0)),
            pl.BlockSpec((1, T, 1), lambda b: (b, 0, 0)),
        ],
        out_shape=[
            jax.ShapeDtypeStruct((nb, T, COUT), jnp.float32),
            jax.ShapeDtypeStruct((nb, T, 1), jnp.int32),
        ],
    )(q3, h3, idx3, bern3, jnp.asarray(_POS), mask_emb,
      g2.reshape(1, COUT), be2.reshape(1, COUT))


def kernel(img, W1, b1, g1, be1, codebook, Wp, bp, mask_emb, g2, be2):
    img4 = img.reshape(B, CIN, 1024)

    tbl, cbext = _codebook_xform(codebook, Wp, bp)

    bern = jax.random.bernoulli(jax.random.key(42), MASK_PROB,
                                (B, 1, 1)).astype(jnp.float32)
    bern3 = jnp.broadcast_to(bern, (B, T, 1))

    h4, idx3d = _encode_vq(img4, W1, b1, g1, be1, cbext)
    q = _gather_rows(tbl, idx3d.reshape(NTOK))
    xqo, lab = _finish(q.reshape(B, T, 2 * COUT), h4.reshape(B, T, COUT),
                       idx3d, bern3, mask_emb, g2, be2)
    vm = jnp.ones((B, T), jnp.int32)
    return (xqo, vm, lab.reshape(B, T))
